# Initial kernel scaffold; baseline (speedup 1.0000x reference)
import jax
import jax.numpy as jnp
from jax.experimental import pallas as pl


def kernel(x, edge_index, W1, b1, W2, b2, W3, b3):
    raise NotImplementedError("write your pallas kernel here")


# speedup vs baseline: 7.7380x; 7.7380x over previous
import functools

import jax
import jax.numpy as jnp
from jax import lax
from jax.experimental import pallas as pl
from jax.experimental.pallas import tpu as pltpu
from jax.experimental.pallas import tpu_sc as plsc

N = 10000
E = 320000
D = 128
NC = 2
NS = 16
NW = NC * NS
EB = 128
NB = 80
GB = 8
NG = NB // GB
E_PAD = NW * NB * EB
N_PAD = 10112
ROWS = N_PAD // NS
NBUF = 2
DW = 16

_MESH = plsc.VectorSubcoreMesh(core_axis_name="c", subcore_axis_name="s")


def _sc_degree(dst3, zeros16, ones16):
    @functools.partial(
        pl.kernel,
        out_type=jax.ShapeDtypeStruct((NC, N_PAD, DW), jnp.float32),
        mesh=_MESH,
        scratch_types=[
            pltpu.VMEM_SHARED((N_PAD, DW), jnp.float32),
            pltpu.VMEM((NB, EB), jnp.int32),
            pltpu.VMEM((EB, DW), jnp.float32),
        ],
    )
    def run(dst_hbm, zero_hbm, ones_hbm, out_hbm, acc, dst_v, ones_v):
        c = lax.axis_index("c")
        s = lax.axis_index("s")
        w = c * NS + s
        pltpu.sync_copy(zero_hbm.at[pl.ds(s * ROWS, ROWS)],
                        acc.at[pl.ds(s * ROWS, ROWS)])
        pltpu.sync_copy(dst_hbm.at[w], dst_v)
        pltpu.sync_copy(ones_hbm, ones_v)
        plsc.subcore_barrier()

        @pl.loop(0, NB)
        def _batches(j):
            pltpu.sync_copy(ones_v, acc.at[dst_v.at[j]], add=True)

        plsc.subcore_barrier()
        pltpu.sync_copy(acc.at[pl.ds(s * ROWS, ROWS)],
                        out_hbm.at[c, pl.ds(s * ROWS, ROWS)])

    return run(dst3, zeros16, ones16)


def _sc_propagate(p, src3, dst3, zeros):
    @functools.partial(
        pl.kernel,
        out_type=jax.ShapeDtypeStruct((NC, N_PAD, D), jnp.float32),
        mesh=_MESH,
        scratch_types=[
            pltpu.VMEM_SHARED((N_PAD, D), jnp.float32),
            pltpu.VMEM((NB, EB), jnp.int32),
            pltpu.VMEM((2, GB, EB), jnp.int32),
            pltpu.VMEM((NBUF, EB, D), jnp.float32),
            pltpu.SemaphoreType.DMA((NBUF,)),
            pltpu.SemaphoreType.DMA((NBUF,)),
            pltpu.SemaphoreType.DMA((2,)),
        ],
    )
    def run(p_hbm, src_hbm, dst_hbm, zero_hbm, out_hbm,
            acc, src_v, dstb, rows, gsem, ssem, dgsem):
        c = lax.axis_index("c")
        s = lax.axis_index("s")
        w = c * NS + s
        pltpu.sync_copy(zero_hbm.at[pl.ds(s * ROWS, ROWS)],
                        acc.at[pl.ds(s * ROWS, ROWS)])
        pltpu.sync_copy(src_hbm.at[w], src_v)
        plsc.subcore_barrier()

        for g in range(2):
            pltpu.async_copy(dst_hbm.at[w, pl.ds(g * GB, GB)], dstb.at[g],
                             dgsem.at[g])
        for b in range(NBUF):
            pltpu.async_copy(p_hbm.at[src_v.at[b]], rows.at[b], gsem.at[b])

        @pl.loop(0, NG)
        def _groups(g):
            gmod = lax.rem(g, 2)
            pltpu.make_async_copy(dst_hbm.at[w, pl.ds(g * GB, GB)],
                                  dstb.at[gmod], dgsem.at[gmod]).wait()
            for b8 in range(GB):
                j = g * GB + b8
                b = b8 % NBUF
                pltpu.make_async_copy(p_hbm.at[src_v.at[j]], rows.at[b],
                                      gsem.at[b]).wait()
                pltpu.async_copy(rows.at[b], acc.at[dstb.at[gmod, b8]],
                                 ssem.at[b], add=True)
                pltpu.make_async_copy(rows.at[b], acc.at[dstb.at[gmod, b8]],
                                      ssem.at[b]).wait()

                @pl.when(j + NBUF < NB)
                def _next():
                    pltpu.async_copy(p_hbm.at[src_v.at[j + NBUF]],
                                     rows.at[b], gsem.at[b])

            @pl.when(g + 2 < NG)
            def _next_group():
                pltpu.async_copy(dst_hbm.at[w, pl.ds((g + 2) * GB, GB)],
                                 dstb.at[gmod], dgsem.at[gmod])

        plsc.subcore_barrier()
        pltpu.sync_copy(acc.at[pl.ds(s * ROWS, ROWS)],
                        out_hbm.at[c, pl.ds(s * ROWS, ROWS)])

    return run(p, src3, dst3, zeros)


BN = 1000


def _dinv_block(d0_ref, d1_ref):
    cnt = d0_ref[:, 0:1] + d1_ref[:, 0:1]
    return lax.rsqrt(cnt + 1.0)


def _tc_first(x, W, d0, d1):
    def body(x_ref, w_ref, d0_ref, d1_ref, o_ref):
        dinv = _dinv_block(d0_ref, d1_ref)
        h = jnp.dot(x_ref[:, :], w_ref[:, :],
                    preferred_element_type=jnp.float32)
        o_ref[:, :] = h * dinv

    return pl.pallas_call(
        body,
        grid=(N // BN,),
        in_specs=[
            pl.BlockSpec((BN, D), lambda i: (i, 0)),
            pl.BlockSpec((D, D), lambda i: (0, 0)),
            pl.BlockSpec((BN, DW), lambda i: (i, 0)),
            pl.BlockSpec((BN, DW), lambda i: (i, 0)),
        ],
        out_specs=pl.BlockSpec((BN, D), lambda i: (i, 0)),
        out_shape=jax.ShapeDtypeStruct((N, D), jnp.float32),
    )(x, W, d0, d1)


def _tc_mid(part0, part1, pprev, d0, d1, bvec, W):
    def body(p0_ref, p1_ref, pp_ref, d0_ref, d1_ref, b_ref, w_ref, o_ref):
        dinv = _dinv_block(d0_ref, d1_ref)
        sacc = p0_ref[:, :] + p1_ref[:, :] + pp_ref[:, :]
        z = jnp.maximum(sacc * dinv + b_ref[:, :], 0.0)
        h = jnp.dot(z, w_ref[:, :], preferred_element_type=jnp.float32)
        o_ref[:, :] = h * dinv

    return pl.pallas_call(
        body,
        grid=(N // BN,),
        in_specs=[
            pl.BlockSpec((BN, D), lambda i: (i, 0)),
            pl.BlockSpec((BN, D), lambda i: (i, 0)),
            pl.BlockSpec((BN, D), lambda i: (i, 0)),
            pl.BlockSpec((BN, DW), lambda i: (i, 0)),
            pl.BlockSpec((BN, DW), lambda i: (i, 0)),
            pl.BlockSpec((1, D), lambda i: (0, 0)),
            pl.BlockSpec((D, D), lambda i: (0, 0)),
        ],
        out_specs=pl.BlockSpec((BN, D), lambda i: (i, 0)),
        out_shape=jax.ShapeDtypeStruct((N, D), jnp.float32),
    )(part0, part1, pprev, d0, d1, bvec, W)


def _tc_final(part0, part1, pprev, d0, d1, bvec):
    def body(p0_ref, p1_ref, pp_ref, d0_ref, d1_ref, b_ref, o_ref):
        dinv = _dinv_block(d0_ref, d1_ref)
        sacc = p0_ref[:, :] + p1_ref[:, :] + pp_ref[:, :]
        o_ref[:, :] = sacc * dinv + b_ref[:, :]

    return pl.pallas_call(
        body,
        grid=(N // BN,),
        in_specs=[
            pl.BlockSpec((BN, D), lambda i: (i, 0)),
            pl.BlockSpec((BN, D), lambda i: (i, 0)),
            pl.BlockSpec((BN, D), lambda i: (i, 0)),
            pl.BlockSpec((BN, DW), lambda i: (i, 0)),
            pl.BlockSpec((BN, DW), lambda i: (i, 0)),
            pl.BlockSpec((1, D), lambda i: (0, 0)),
        ],
        out_specs=pl.BlockSpec((BN, D), lambda i: (i, 0)),
        out_shape=jax.ShapeDtypeStruct((N, D), jnp.float32),
    )(part0, part1, pprev, d0, d1, bvec)


def kernel(x, edge_index, W1, b1, W2, b2, W3, b3):
    pad = E_PAD - E
    src3 = jnp.concatenate(
        [edge_index[0], jnp.zeros((pad,), jnp.int32)]).reshape(NW, NB, EB)
    dst3 = jnp.concatenate(
        [edge_index[1], jnp.full((pad,), N, jnp.int32)]).reshape(NW, NB, EB)
    zeros = jnp.zeros((N_PAD, D), jnp.float32)
    zeros16 = jnp.zeros((N_PAD, DW), jnp.float32)
    ones16 = jnp.ones((EB, DW), jnp.float32)

    deg = _sc_degree(dst3, zeros16, ones16)
    d0, d1 = deg[0], deg[1]
    b1r, b2r, b3r = (b.reshape(1, D) for b in (b1, b2, b3))

    p1 = _tc_first(x, W1, d0, d1)
    parts = _sc_propagate(p1, src3, dst3, zeros)
    p2 = _tc_mid(parts[0], parts[1], p1, d0, d1, b1r, W2)
    parts = _sc_propagate(p2, src3, dst3, zeros)
    p3 = _tc_mid(parts[0], parts[1], p2, d0, d1, b2r, W3)
    parts = _sc_propagate(p3, src3, dst3, zeros)
    return _tc_final(parts[0], parts[1], p3, d0, d1, b3r)


# speedup vs baseline: 24.5633x; 3.1744x over previous
import functools

import jax
import jax.numpy as jnp
from jax import lax
from jax.experimental import pallas as pl
from jax.experimental.pallas import tpu as pltpu
from jax.experimental.pallas import tpu_sc as plsc

N = 10000
E = 320000
D = 128
NC = 2
NS = 16
NW = NC * NS
EB = 128
NB = 80
GB = 8
NG = NB // GB
E_PAD = NW * NB * EB
N_PAD = 10112
ROWS = N_PAD // NS
NBUF = 2
DW = 16

_MESH = plsc.VectorSubcoreMesh(core_axis_name="c", subcore_axis_name="s")


def _sc_degree(dst3, zeros16, ones16):
    @functools.partial(
        pl.kernel,
        out_type=jax.ShapeDtypeStruct((NC, N_PAD, DW), jnp.float32),
        mesh=_MESH,
        scratch_types=[
            pltpu.VMEM_SHARED((N_PAD, DW), jnp.float32),
            pltpu.VMEM((NB, EB), jnp.int32),
            pltpu.VMEM((EB, DW), jnp.float32),
        ],
    )
    def run(dst_hbm, zero_hbm, ones_hbm, out_hbm, acc, dst_v, ones_v):
        c = lax.axis_index("c")
        s = lax.axis_index("s")
        w = c * NS + s
        pltpu.sync_copy(zero_hbm.at[pl.ds(s * ROWS, ROWS)],
                        acc.at[pl.ds(s * ROWS, ROWS)])
        pltpu.sync_copy(dst_hbm.at[w], dst_v)
        pltpu.sync_copy(ones_hbm, ones_v)
        plsc.subcore_barrier()

        @pl.loop(0, NB)
        def _batches(j):
            pltpu.sync_copy(ones_v, acc.at[dst_v.at[j]], add=True)

        plsc.subcore_barrier()
        pltpu.sync_copy(acc.at[pl.ds(s * ROWS, ROWS)],
                        out_hbm.at[c, pl.ds(s * ROWS, ROWS)])

    return run(dst3, zeros16, ones16)


def _sc_propagate(p, src3, dst3, zeros):
    @functools.partial(
        pl.kernel,
        out_type=jax.ShapeDtypeStruct((NC, N_PAD, D), jnp.float32),
        mesh=_MESH,
        scratch_types=[
            pltpu.VMEM_SHARED((N_PAD, D), jnp.float32),
            pltpu.VMEM((NB, EB), jnp.int32),
            pltpu.VMEM((2, GB, EB), jnp.int32),
            pltpu.VMEM((NBUF, EB, D), jnp.float32),
            pltpu.SemaphoreType.DMA((NBUF,)),
            pltpu.SemaphoreType.DMA((NBUF,)),
            pltpu.SemaphoreType.DMA((2,)),
        ],
    )
    def run(p_hbm, src_hbm, dst_hbm, zero_hbm, out_hbm,
            acc, src_v, dstb, rows, gsem, ssem, dgsem):
        c = lax.axis_index("c")
        s = lax.axis_index("s")
        w = c * NS + s
        pltpu.sync_copy(zero_hbm.at[pl.ds(s * ROWS, ROWS)],
                        acc.at[pl.ds(s * ROWS, ROWS)])
        pltpu.sync_copy(src_hbm.at[w], src_v)
        plsc.subcore_barrier()

        for g in range(2):
            pltpu.async_copy(dst_hbm.at[w, pl.ds(g * GB, GB)], dstb.at[g],
                             dgsem.at[g])
        for b in range(NBUF):
            pltpu.async_copy(p_hbm.at[src_v.at[b]], rows.at[b], gsem.at[b])

        @pl.loop(0, NG)
        def _groups(g):
            gmod = lax.rem(g, 2)
            pltpu.make_async_copy(dst_hbm.at[w, pl.ds(g * GB, GB)],
                                  dstb.at[gmod], dgsem.at[gmod]).wait()
            for b8 in range(GB):
                j = g * GB + b8
                b = b8 % NBUF
                pltpu.make_async_copy(p_hbm.at[src_v.at[j]], rows.at[b],
                                      gsem.at[b]).wait()
                pltpu.async_copy(rows.at[b], acc.at[dstb.at[gmod, b8]],
                                 ssem.at[b], add=True)
                pltpu.make_async_copy(rows.at[b], acc.at[dstb.at[gmod, b8]],
                                      ssem.at[b]).wait()

                @pl.when(j + NBUF < NB)
                def _next():
                    pltpu.async_copy(p_hbm.at[src_v.at[j + NBUF]],
                                     rows.at[b], gsem.at[b])

            @pl.when(g + 2 < NG)
            def _next_group():
                pltpu.async_copy(dst_hbm.at[w, pl.ds((g + 2) * GB, GB)],
                                 dstb.at[gmod], dgsem.at[gmod])

        plsc.subcore_barrier()
        pltpu.sync_copy(acc.at[pl.ds(s * ROWS, ROWS)],
                        out_hbm.at[c, pl.ds(s * ROWS, ROWS)])

    return run(p, src3, dst3, zeros)


BN = 1000


def _dinv_block(d0_ref, d1_ref):
    cnt = d0_ref[:, 0:1] + d1_ref[:, 0:1]
    return lax.rsqrt(cnt + 1.0)


def _tc_first(x, W, d0, d1):
    def body(x_ref, w_ref, d0_ref, d1_ref, o_ref):
        dinv = _dinv_block(d0_ref, d1_ref)
        h = jnp.dot(x_ref[:, :], w_ref[:, :],
                    preferred_element_type=jnp.float32)
        o_ref[:, :] = h * dinv

    return pl.pallas_call(
        body,
        grid=(N // BN,),
        in_specs=[
            pl.BlockSpec((BN, D), lambda i: (i, 0)),
            pl.BlockSpec((D, D), lambda i: (0, 0)),
            pl.BlockSpec((BN, DW), lambda i: (i, 0)),
            pl.BlockSpec((BN, DW), lambda i: (i, 0)),
        ],
        out_specs=pl.BlockSpec((BN, D), lambda i: (i, 0)),
        out_shape=jax.ShapeDtypeStruct((N, D), jnp.float32),
    )(x, W, d0, d1)


def _tc_mid(part0, part1, pprev, d0, d1, bvec, W):
    def body(p0_ref, p1_ref, pp_ref, d0_ref, d1_ref, b_ref, w_ref, o_ref):
        dinv = _dinv_block(d0_ref, d1_ref)
        sacc = p0_ref[:, :] + p1_ref[:, :] + pp_ref[:, :]
        z = jnp.maximum(sacc * dinv + b_ref[:, :], 0.0)
        h = jnp.dot(z, w_ref[:, :], preferred_element_type=jnp.float32)
        o_ref[:, :] = h * dinv

    return pl.pallas_call(
        body,
        grid=(N // BN,),
        in_specs=[
            pl.BlockSpec((BN, D), lambda i: (i, 0)),
            pl.BlockSpec((BN, D), lambda i: (i, 0)),
            pl.BlockSpec((BN, D), lambda i: (i, 0)),
            pl.BlockSpec((BN, DW), lambda i: (i, 0)),
            pl.BlockSpec((BN, DW), lambda i: (i, 0)),
            pl.BlockSpec((1, D), lambda i: (0, 0)),
            pl.BlockSpec((D, D), lambda i: (0, 0)),
        ],
        out_specs=pl.BlockSpec((BN, D), lambda i: (i, 0)),
        out_shape=jax.ShapeDtypeStruct((N, D), jnp.float32),
    )(part0, part1, pprev, d0, d1, bvec, W)


def _tc_final(part0, part1, pprev, d0, d1, bvec):
    def body(p0_ref, p1_ref, pp_ref, d0_ref, d1_ref, b_ref, o_ref):
        dinv = _dinv_block(d0_ref, d1_ref)
        sacc = p0_ref[:, :] + p1_ref[:, :] + pp_ref[:, :]
        o_ref[:, :] = sacc * dinv + b_ref[:, :]

    return pl.pallas_call(
        body,
        grid=(N // BN,),
        in_specs=[
            pl.BlockSpec((BN, D), lambda i: (i, 0)),
            pl.BlockSpec((BN, D), lambda i: (i, 0)),
            pl.BlockSpec((BN, D), lambda i: (i, 0)),
            pl.BlockSpec((BN, DW), lambda i: (i, 0)),
            pl.BlockSpec((BN, DW), lambda i: (i, 0)),
            pl.BlockSpec((1, D), lambda i: (0, 0)),
        ],
        out_specs=pl.BlockSpec((BN, D), lambda i: (i, 0)),
        out_shape=jax.ShapeDtypeStruct((N, D), jnp.float32),
    )(part0, part1, pprev, d0, d1, bvec)


def kernel(x, edge_index, W1, b1, W2, b2, W3, b3):
    pad = E_PAD - E
    pad_iota = jnp.arange(pad, dtype=jnp.int32)
    src3 = jnp.concatenate(
        [edge_index[0], pad_iota % N]).reshape(NW, NB, EB)
    dst3 = jnp.concatenate(
        [edge_index[1], N + pad_iota % (N_PAD - N)]).reshape(NW, NB, EB)
    zeros = jnp.zeros((N_PAD, D), jnp.float32)
    zeros16 = jnp.zeros((N_PAD, DW), jnp.float32)
    ones16 = jnp.ones((EB, DW), jnp.float32)

    deg = _sc_degree(dst3, zeros16, ones16)
    d0, d1 = deg[0], deg[1]
    b1r, b2r, b3r = (b.reshape(1, D) for b in (b1, b2, b3))

    p1 = _tc_first(x, W1, d0, d1)
    parts = _sc_propagate(p1, src3, dst3, zeros)
    p2 = _tc_mid(parts[0], parts[1], p1, d0, d1, b1r, W2)
    parts = _sc_propagate(p2, src3, dst3, zeros)
    p3 = _tc_mid(parts[0], parts[1], p2, d0, d1, b2r, W3)
    parts = _sc_propagate(p3, src3, dst3, zeros)
    return _tc_final(parts[0], parts[1], p3, d0, d1, b3r)
